# baseline (speedup 1.0000x reference)
import functools

import jax
import jax.numpy as jnp
from jax import lax
from jax.experimental import pallas as pl
from jax.experimental.pallas import tpu as pltpu
from jax.experimental.pallas import tpu_sc as plsc

_B, _D, _H, _W = 8, 256, 32, 32
_HW = _H * _W
_NW = 32
_RPW = (2 * _D) // _NW
_CHUNK = _RPW * _HW


def _sc_body(row_hbm, col_hbm, out_hbm, tab_ref, chunk_ref, sem):
    wid = lax.axis_index("c") * 16 + lax.axis_index("s")
    c0 = wid * _RPW

    pltpu.sync_copy(col_hbm.at[pl.ds(0, _W), :], tab_ref.at[pl.ds(0, 32), :])
    pltpu.sync_copy(row_hbm.at[pl.ds(0, _H), :], tab_ref.at[pl.ds(32, 32), :])

    iota16 = lax.broadcasted_iota(jnp.int32, (16,), 0)

    @pl.when(wid < 16)
    def _build_col_rows():
        def per_row(i, _):
            c = c0 + i
            idx_col = jnp.full((16,), c, jnp.int32)
            va = plsc.load_gather(tab_ref, [iota16, idx_col])
            vb = plsc.load_gather(tab_ref, [iota16 + 16, idx_col])
            base = i * _HW

            def store_j(j, _):
                for u in range(4):
                    off = base + (j * 4 + u) * 32
                    chunk_ref[pl.ds(off, 16)] = va
                    chunk_ref[pl.ds(off + 16, 16)] = vb
                return 0

            lax.fori_loop(0, 8, store_j, 0)
            return 0

        lax.fori_loop(0, _RPW, per_row, 0)

    @pl.when(wid >= 16)
    def _build_row_rows():
        def per_row(i, _):
            c = c0 - 2 * _D + _D + i
            idx_col = jnp.full((16,), c, jnp.int32)
            base = i * _HW

            def store_j(j, _):
                for u in range(2):
                    jj = j * 2 + u
                    idx_row = jnp.full((16,), 32 + jj, jnp.int32)
                    v = plsc.load_gather(tab_ref, [idx_row, idx_col])
                    chunk_ref[pl.ds(base + jj * 32, 16)] = v
                    chunk_ref[pl.ds(base + jj * 32 + 16, 16)] = v
                return 0

            lax.fori_loop(0, 16, store_j, 0)
            return 0

        lax.fori_loop(0, _RPW, per_row, 0)

    copies = [
        pltpu.async_copy(
            chunk_ref, out_hbm.at[b, pl.ds(c0 * _HW, _CHUNK)], sem)
        for b in range(_B)
    ]
    for c in copies:
        c.wait()


def kernel(pixel_values, row_embeddings, column_embeddings):
    B = pixel_values.shape[0]
    H = pixel_values.shape[-2]
    W = pixel_values.shape[-1]
    D = row_embeddings.shape[-1]
    mesh = plsc.VectorSubcoreMesh(core_axis_name="c", subcore_axis_name="s")
    run = functools.partial(
        pl.kernel,
        out_type=jax.ShapeDtypeStruct((B, 2 * D * H * W), jnp.float32),
        mesh=mesh,
        scratch_types=[
            pltpu.VMEM((64, D), jnp.float32),
            pltpu.VMEM((_CHUNK,), jnp.float32),
            pltpu.SemaphoreType.DMA,
        ],
        compiler_params=pltpu.CompilerParams(needs_layout_passes=False),
    )(_sc_body)
    out = run(row_embeddings, column_embeddings)
    return out.reshape(B, 2 * D, H, W)


# speedup vs baseline: 1.0304x; 1.0304x over previous
import functools

import jax
import jax.numpy as jnp
from jax import lax
from jax.experimental import pallas as pl
from jax.experimental.pallas import tpu as pltpu
from jax.experimental.pallas import tpu_sc as plsc

_B, _D, _H, _W = 8, 256, 32, 32
_HW = _H * _W
_NW = 32
_RPW = (2 * _D) // _NW
_CHUNK = _RPW * _HW


def _sc_body(row_hbm, col_hbm, out_hbm, tab_ref, chunk_ref, sem):
    wid = lax.axis_index("c") * 16 + lax.axis_index("s")
    c0 = wid * _RPW

    pltpu.sync_copy(col_hbm.at[pl.ds(0, _W), :], tab_ref.at[pl.ds(0, 32), :])
    pltpu.sync_copy(row_hbm.at[pl.ds(0, _H), :], tab_ref.at[pl.ds(32, 32), :])

    iota16 = lax.broadcasted_iota(jnp.int32, (16,), 0)

    @pl.when(wid < 16)
    def _build_col_rows():
        def per_row(i, _):
            c = c0 + i
            idx_col = jnp.full((16,), c, jnp.int32)
            va = plsc.load_gather(tab_ref, [iota16, idx_col])
            vb = plsc.load_gather(tab_ref, [iota16 + 16, idx_col])
            base = i * _HW

            def store_j(j, _):
                for u in range(4):
                    off = base + (j * 4 + u) * 32
                    chunk_ref[pl.ds(off, 16)] = va
                    chunk_ref[pl.ds(off + 16, 16)] = vb
                return 0

            lax.fori_loop(0, 8, store_j, 0)
            return 0

        lax.fori_loop(0, _RPW, per_row, 0)

    @pl.when(wid >= 16)
    def _build_row_rows():
        def per_row(i, _):
            c = c0 - 2 * _D + _D + i
            idx_col = jnp.full((16,), c, jnp.int32)
            base = i * _HW

            def store_j(j, _):
                for u in range(2):
                    jj = j * 2 + u
                    idx_row = jnp.full((16,), 32 + jj, jnp.int32)
                    v = plsc.load_gather(tab_ref, [idx_row, idx_col])
                    chunk_ref[pl.ds(base + jj * 32, 16)] = v
                    chunk_ref[pl.ds(base + jj * 32 + 16, 16)] = v
                return 0

            lax.fori_loop(0, 16, store_j, 0)
            return 0

        lax.fori_loop(0, _RPW, per_row, 0)

    copies = [
        pltpu.async_copy(
            chunk_ref, out_hbm.at[b, pl.ds(c0 * _HW, _CHUNK)], sem)
        for b in range(_B)
    ]
    for c in copies:
        c.wait()


def kernel(pixel_values, row_embeddings, column_embeddings):
    B = pixel_values.shape[0]
    H = pixel_values.shape[-2]
    W = pixel_values.shape[-1]
    D = row_embeddings.shape[-1]
    mesh = plsc.VectorSubcoreMesh(core_axis_name="c", subcore_axis_name="s")
    run = functools.partial(
        pl.kernel,
        out_type=jax.ShapeDtypeStruct((B, 2 * D * H * W), jnp.float32),
        mesh=mesh,
        scratch_types=[
            pltpu.VMEM((64, D), jnp.float32),
            pltpu.VMEM((_CHUNK,), jnp.float32),
            pltpu.SemaphoreType.DMA,
        ],
        compiler_params=pltpu.CompilerParams(
            needs_layout_passes=False, use_tc_tiling_on_sc=True),
    )(_sc_body)
    out = run(row_embeddings, column_embeddings)
    return out.reshape(B, 2 * D, H, W)


# speedup vs baseline: 2.4982x; 2.4244x over previous
import functools

import jax
import jax.numpy as jnp
from jax import lax
from jax.experimental import pallas as pl
from jax.experimental.pallas import tpu as pltpu
from jax.experimental.pallas import tpu_sc as plsc

_B, _D, _H, _W = 8, 256, 32, 32
_HW = _H * _W
_NW = 32
_RPW = (2 * _D) // _NW
_CHUNK = _RPW * _HW


def _sc_body(row_hbm, col_hbm, out_hbm, tab_ref, chunk_ref, sem):
    wid = lax.axis_index("c") * 16 + lax.axis_index("s")
    c0 = wid * _RPW

    pltpu.sync_copy(col_hbm.at[pl.ds(0, _W), :], tab_ref.at[pl.ds(0, 32), :])
    pltpu.sync_copy(row_hbm.at[pl.ds(0, _H), :], tab_ref.at[pl.ds(32, 32), :])

    iota16 = lax.broadcasted_iota(jnp.int32, (16,), 0)

    @pl.when(wid < 16)
    def _build_col_rows():
        def per_row(i, _):
            c = c0 + i
            idx_col = jnp.full((16,), c, jnp.int32)
            va = plsc.load_gather(tab_ref, [iota16, idx_col])
            vb = plsc.load_gather(tab_ref, [iota16 + 16, idx_col])
            base = i * _HW

            def store_j(j, _):
                for u in range(4):
                    off = (j * 4 + u) * 32
                    chunk_ref[i, pl.ds(off, 16)] = va
                    chunk_ref[i, pl.ds(off + 16, 16)] = vb
                return 0

            lax.fori_loop(0, 8, store_j, 0)
            return 0

        lax.fori_loop(0, _RPW, per_row, 0)

    @pl.when(wid >= 16)
    def _build_row_rows():
        def per_row(i, _):
            c = c0 - 2 * _D + _D + i
            idx_col = jnp.full((16,), c, jnp.int32)

            def store_j(j, _):
                for u in range(2):
                    jj = j * 2 + u
                    idx_row = jnp.full((16,), 32 + jj, jnp.int32)
                    v = plsc.load_gather(tab_ref, [idx_row, idx_col])
                    chunk_ref[i, pl.ds(jj * 32, 16)] = v
                    chunk_ref[i, pl.ds(jj * 32 + 16, 16)] = v
                return 0

            lax.fori_loop(0, 16, store_j, 0)
            return 0

        lax.fori_loop(0, _RPW, per_row, 0)

    copies = [
        pltpu.async_copy(
            chunk_ref, out_hbm.at[b, pl.ds(c0, _RPW), :], sem)
        for b in range(_B)
    ]
    for c in copies:
        c.wait()


def kernel(pixel_values, row_embeddings, column_embeddings):
    B = pixel_values.shape[0]
    H = pixel_values.shape[-2]
    W = pixel_values.shape[-1]
    D = row_embeddings.shape[-1]
    mesh = plsc.VectorSubcoreMesh(core_axis_name="c", subcore_axis_name="s")
    run = functools.partial(
        pl.kernel,
        out_type=jax.ShapeDtypeStruct((B, 2 * D, H * W), jnp.float32),
        mesh=mesh,
        scratch_types=[
            pltpu.VMEM((64, D), jnp.float32),
            pltpu.VMEM((_RPW, _HW), jnp.float32),
            pltpu.SemaphoreType.DMA,
        ],
        compiler_params=pltpu.CompilerParams(
            needs_layout_passes=False, use_tc_tiling_on_sc=True),
    )(_sc_body)
    out = run(row_embeddings, column_embeddings)
    return out.reshape(B, 2 * D, H, W)


# speedup vs baseline: 4.8081x; 1.9246x over previous
import jax
import jax.numpy as jnp
from jax import lax
from jax.experimental import pallas as pl
from jax.experimental.pallas import tpu as pltpu

_B, _D, _H, _W = 8, 256, 32, 32
_ROWS_PER_BLOCK = 128
_BLOCKS_PER_PLANE = (2 * _D) // _ROWS_PER_BLOCK


def _pos_embed_kernel(row_ref, col_ref, out_ref, big_ref, sem):
    col = col_ref[:_W, :]
    row = row_ref[:_H, :]
    k = lax.broadcasted_iota(jnp.int32, (_W, _H * _W), 0)
    hw = lax.broadcasted_iota(jnp.int32, (_W, _H * _W), 1)
    sel_w = (hw % _W == k).astype(jnp.float32)
    sel_h = (hw // _W == k).astype(jnp.float32)
    dn = (((0,), (0,)), ((), ()))
    x_part = lax.dot_general(col, sel_w, dn, preferred_element_type=jnp.float32)
    y_part = lax.dot_general(row, sel_h, dn, preferred_element_type=jnp.float32)
    for b in range(_B):
        big_ref[b, :_D, :] = x_part
        big_ref[b, _D:, :] = y_part
    copies = [
        pltpu.async_copy(big_ref.at[b], out_ref.at[b], sem, priority=b % 2)
        for b in range(_B)
    ]
    for c in copies:
        c.wait()


def kernel(pixel_values, row_embeddings, column_embeddings):
    B = pixel_values.shape[0]
    H = pixel_values.shape[-2]
    W = pixel_values.shape[-1]
    D = row_embeddings.shape[-1]
    out = pl.pallas_call(
        _pos_embed_kernel,
        in_specs=[
            pl.BlockSpec(memory_space=pltpu.VMEM),
            pl.BlockSpec(memory_space=pltpu.VMEM),
        ],
        out_specs=pl.BlockSpec(memory_space=pl.ANY),
        out_shape=jax.ShapeDtypeStruct((B, 2 * D, H * W), jnp.float32),
        scratch_shapes=[
            pltpu.VMEM((B, 2 * D, H * W), jnp.float32),
            pltpu.SemaphoreType.DMA,
        ],
    )(row_embeddings, column_embeddings)
    return out.reshape(B, 2 * D, H, W)
